# baseline (speedup 1.0000x reference)
import functools

import jax
import jax.numpy as jnp
import numpy as np
from jax import lax
from jax.experimental import pallas as pl
from jax.experimental.pallas import tpu as pltpu
from jax.experimental.pallas import tpu_sc as plsc

N_POINTS = 262144
N_LEVELS = 12
F_PER_LEVEL = 2
LOG2_T = 19
TABLE_SIZE = 2 ** LOG2_T
MASK = TABLE_SIZE - 1
BASE_RES = 16
LEVEL_SCALE = 1.38
WIDTH = 64
OUT_DIM = 16
IN_DIM = N_LEVELS * F_PER_LEVEL
RES = [float(np.floor(BASE_RES * LEVEL_SCALE ** l)) for l in range(N_LEVELS)]
P1 = np.int32(np.uint32(2654435761).astype(np.int32))
P2 = np.int32(805459861)

NC, NS, LANES = 2, 16, 16
NW = NC * NS
NP = N_POINTS // NW
CHUNK = 128
NCHUNK = NP // CHUNK
NGROUP = CHUNK // LANES
PE_CHUNK = IN_DIM * CHUNK
CORNERS = [(cx, cy, cz) for cx in (0, 1) for cy in (0, 1) for cz in (0, 1)]

MLP_BN = 2048


def _sc_encode_body(xt_hbm, tab_hbm, norm_hbm, pe_hbm,
                    xb, normv, idxb, tb, peb, *rest):
    rows = (rest[0:16], rest[16:32])
    sems = rest[32:34]
    wid = lax.axis_index("s") * NC + lax.axis_index("c")
    base = wid * NP
    gcb = wid * NCHUNK
    pltpu.sync_copy(norm_hbm, normv)

    def compute_idx(l, p):
        res = jnp.float32(RES[l])
        off = jnp.int32(l * TABLE_SIZE)

        def gbody(i, c):
            s = pl.ds(i * LANES, LANES)
            scx = normv[pl.ds(0 * CHUNK, LANES)] * res
            scy = normv[pl.ds(1 * CHUNK, LANES)] * res
            scz = normv[pl.ds(2 * CHUNK, LANES)] * res
            px = xb[0, s] * scx - normv[pl.ds(3 * CHUNK, LANES)] * scx
            py = xb[1, s] * scy - normv[pl.ds(4 * CHUNK, LANES)] * scy
            pz = xb[2, s] * scz - normv[pl.ds(5 * CHUNK, LANES)] * scz
            bx = px.astype(jnp.int32)
            by = py.astype(jnp.int32)
            bz = pz.astype(jnp.int32)
            tb[3 * p + 0, s] = px - bx.astype(jnp.float32)
            tb[3 * p + 1, s] = py - by.astype(jnp.float32)
            tb[3 * p + 2, s] = pz - bz.astype(jnp.float32)
            a0 = (bx & MASK) | off
            a1 = ((bx + 1) & MASK) | off
            byp = by * P1
            b0 = byp & MASK
            b1 = (byp + P1) & MASK
            bzp = bz * P2
            c0 = bzp & MASK
            c1 = (bzp + P2) & MASK
            av = (a0, a1)
            bv = (b0, b1)
            cv = (c0, c1)
            for ci, (cx, cy, cz) in enumerate(CORNERS):
                e2 = (av[cx] ^ bv[cy] ^ cv[cz]) * 2
                idxb[16 * p + 2 * ci, s] = e2
                idxb[16 * p + 2 * ci + 1, s] = e2 + 1
            return c

        lax.fori_loop(0, NGROUP, gbody, 0, unroll=False)

    def fire(p):
        return [pltpu.async_copy(tab_hbm.at[idxb.at[16 * p + j]],
                                 rows[p][j], sems[p])
                for j in range(16)]

    def interp(l, p):
        def gbody(i, c):
            s = pl.ds(i * LANES, LANES)
            tx = tb[3 * p + 0, s]
            ty = tb[3 * p + 1, s]
            tz = tb[3 * p + 2, s]
            one = jnp.float32(1.0)
            wx = (one - tx, tx)
            wy = (one - ty, ty)
            wz = (one - tz, tz)
            wxy = [[wx[a] * wy[b] for b in (0, 1)] for a in (0, 1)]
            f0 = jnp.zeros((LANES,), jnp.float32)
            f1 = jnp.zeros((LANES,), jnp.float32)
            for ci, (cx, cy, cz) in enumerate(CORNERS):
                w = wxy[cx][cy] * wz[cz]
                f0 = f0 + rows[p][2 * ci][s] * w
                f1 = f1 + rows[p][2 * ci + 1][s] * w
            peb[pl.ds(2 * l * CHUNK + i * LANES, LANES)] = f0
            peb[pl.ds((2 * l + 1) * CHUNK + i * LANES, LANES)] = f1
            return c

        lax.fori_loop(0, NGROUP, gbody, 0, unroll=False)

    def chunk_body(ch, c):
        off = base + ch * CHUNK
        for d in range(3):
            pltpu.sync_copy(xt_hbm.at[pl.ds(d * N_POINTS + off, CHUNK)],
                            xb.at[d])
        compute_idx(0, 0)
        cur = fire(0)
        nxt = None
        for l in range(N_LEVELS):
            p = l & 1
            if l + 1 < N_LEVELS:
                compute_idx(l + 1, 1 - p)
                nxt = fire(1 - p)
            for cp in cur:
                cp.wait()
            interp(l, p)
            cur = nxt
        pltpu.sync_copy(peb, pe_hbm.at[pl.ds((gcb + ch) * PE_CHUNK, PE_CHUNK)])
        return c

    lax.fori_loop(0, NCHUNK, chunk_body, 0, unroll=False)


def _mlp_body(pe_ref, w0_ref, b0_ref, w1_ref, b1_ref, w2_ref, b2_ref, o_ref):
    nb = MLP_BN // CHUNK
    pe = jnp.concatenate([pe_ref[b] for b in range(nb)], axis=-1)
    dn = (((0,), (0,)), ((), ()))
    h = lax.dot_general(w0_ref[...], pe, dn, preferred_element_type=jnp.float32)
    h = jnp.maximum(h + b0_ref[...], 0.0)
    h = lax.dot_general(w1_ref[...], h, dn, preferred_element_type=jnp.float32)
    h = jnp.maximum(h + b1_ref[...], 0.0)
    z = lax.dot_general(w2_ref[...], h, dn, preferred_element_type=jnp.float32)
    z0 = z[0:1, :] + b2_ref[...]
    o_ref[...] = jnp.maximum(z0, 0.0) + jnp.log1p(jnp.exp(-jnp.abs(z0)))


@jax.jit
def kernel(x, bounding_box, tables, W0, b0, W1, b1, W2, b2):
    xt1 = x.T.reshape(-1)
    inv_ext = 1.0 / (bounding_box[1] - bounding_box[0])
    norm1 = jnp.repeat(
        jnp.concatenate([inv_ext, bounding_box[0]]).astype(jnp.float32), CHUNK)
    tabf = tables.reshape(-1)

    mesh = plsc.VectorSubcoreMesh(core_axis_name="c", subcore_axis_name="s",
                                  num_cores=NC, num_subcores=NS)
    pe1 = pl.kernel(
        _sc_encode_body,
        out_type=jax.ShapeDtypeStruct((N_POINTS * IN_DIM,), jnp.float32),
        mesh=mesh,
        scratch_types=(
            [
                pltpu.VMEM((3, CHUNK), jnp.float32),
                pltpu.VMEM((6 * CHUNK,), jnp.float32),
                pltpu.VMEM((32, CHUNK), jnp.int32),
                pltpu.VMEM((6, CHUNK), jnp.float32),
                pltpu.VMEM((PE_CHUNK,), jnp.float32),
            ]
            + [pltpu.VMEM((CHUNK,), jnp.float32)
               for _ in range(32)]
            + [pltpu.SemaphoreType.DMA, pltpu.SemaphoreType.DMA]
        ),
        compiler_params=pltpu.CompilerParams(needs_layout_passes=False,
                                             use_tc_tiling_on_sc=False),
    )(xt1, tabf, norm1)

    pe3 = pe1.reshape(N_POINTS // CHUNK, IN_DIM, CHUNK)
    grid = (N_POINTS // MLP_BN,)
    dens = pl.pallas_call(
        _mlp_body,
        grid=grid,
        in_specs=[
            pl.BlockSpec((MLP_BN // CHUNK, IN_DIM, CHUNK), lambda i: (i, 0, 0)),
            pl.BlockSpec((IN_DIM, WIDTH), lambda i: (0, 0)),
            pl.BlockSpec((WIDTH, 1), lambda i: (0, 0)),
            pl.BlockSpec((WIDTH, WIDTH), lambda i: (0, 0)),
            pl.BlockSpec((WIDTH, 1), lambda i: (0, 0)),
            pl.BlockSpec((WIDTH, OUT_DIM), lambda i: (0, 0)),
            pl.BlockSpec((1, 1), lambda i: (0, 0)),
        ],
        out_specs=pl.BlockSpec((1, MLP_BN), lambda i: (0, i)),
        out_shape=jax.ShapeDtypeStruct((1, N_POINTS), jnp.float32),
    )(pe3, W0, b0[:, None], W1, b1[:, None], W2, b2[0:1, None])
    return dens.reshape(N_POINTS)


# speedup vs baseline: 4.9784x; 4.9784x over previous
import functools

import jax
import jax.numpy as jnp
import numpy as np
from jax import lax
from jax.experimental import pallas as pl
from jax.experimental.pallas import tpu as pltpu
from jax.experimental.pallas import tpu_sc as plsc

N_POINTS = 262144
N_LEVELS = 12
F_PER_LEVEL = 2
LOG2_T = 19
TABLE_SIZE = 2 ** LOG2_T
MASK = TABLE_SIZE - 1
BASE_RES = 16
LEVEL_SCALE = 1.38
WIDTH = 64
OUT_DIM = 16
IN_DIM = N_LEVELS * F_PER_LEVEL
RES = [float(np.floor(BASE_RES * LEVEL_SCALE ** l)) for l in range(N_LEVELS)]
P1 = np.int32(np.uint32(2654435761).astype(np.int32))
P2 = np.int32(805459861)

NC, NS, LANES = 2, 16, 16
NW = NC * NS
NP = N_POINTS // NW
CHUNK = 128
NCHUNK = NP // CHUNK
NGROUP = CHUNK // LANES
PE_CHUNK = IN_DIM * CHUNK
CORNERS = [(cx, cy, cz) for cx in (0, 1) for cy in (0, 1) for cz in (0, 1)]

MLP_BN = 2048


def _sc_encode_body(xt_hbm, tab_hbm, norm_hbm, pe_hbm,
                    xb, normv, idxb, tb, peb, *rest):
    rows = (rest[0:16], rest[16:32])
    sems = rest[32:34]
    wid = lax.axis_index("s") * NC + lax.axis_index("c")
    base = wid * NP
    gcb = wid * NCHUNK
    pltpu.sync_copy(norm_hbm, normv)

    def compute_idx(l, p):
        res = jnp.float32(RES[l])
        off = jnp.int32(l * TABLE_SIZE * F_PER_LEVEL)

        def gbody(i, c):
            s = pl.ds(i * LANES, LANES)
            scx = normv[pl.ds(0 * CHUNK, LANES)] * res
            scy = normv[pl.ds(1 * CHUNK, LANES)] * res
            scz = normv[pl.ds(2 * CHUNK, LANES)] * res
            px = xb[0, s] * scx - normv[pl.ds(3 * CHUNK, LANES)] * scx
            py = xb[1, s] * scy - normv[pl.ds(4 * CHUNK, LANES)] * scy
            pz = xb[2, s] * scz - normv[pl.ds(5 * CHUNK, LANES)] * scz
            bx = px.astype(jnp.int32)
            by = py.astype(jnp.int32)
            bz = pz.astype(jnp.int32)
            tb[3 * p + 0, s] = px - bx.astype(jnp.float32)
            tb[3 * p + 1, s] = py - by.astype(jnp.float32)
            tb[3 * p + 2, s] = pz - bz.astype(jnp.float32)
            a0 = bx & MASK
            a1 = (bx + 1) & MASK
            byp = by * P1
            b0 = byp & MASK
            b1 = (byp + P1) & MASK
            bzp = bz * P2
            c0 = bzp & MASK
            c1 = (bzp + P2) & MASK
            av = (a0, a1)
            bv = (b0, b1)
            cv = (c0, c1)
            for ci, (cx, cy, cz) in enumerate(CORNERS):
                e = av[cx] ^ bv[cy] ^ cv[cz]
                g = (((e & -128) << 1) | (e & 127)) | off
                idxb[16 * p + 2 * ci, s] = g
                idxb[16 * p + 2 * ci + 1, s] = g | 128
            return c

        lax.fori_loop(0, NGROUP, gbody, 0, unroll=False)

    def fire(p):
        return [pltpu.async_copy(tab_hbm.at[idxb.at[16 * p + j]],
                                 rows[p][j], sems[p])
                for j in range(16)]

    def interp(l, p):
        def gbody(i, c):
            s = pl.ds(i * LANES, LANES)
            tx = tb[3 * p + 0, s]
            ty = tb[3 * p + 1, s]
            tz = tb[3 * p + 2, s]
            one = jnp.float32(1.0)
            wx = (one - tx, tx)
            wy = (one - ty, ty)
            wz = (one - tz, tz)
            wxy = [[wx[a] * wy[b] for b in (0, 1)] for a in (0, 1)]
            f0 = jnp.zeros((LANES,), jnp.float32)
            f1 = jnp.zeros((LANES,), jnp.float32)
            for ci, (cx, cy, cz) in enumerate(CORNERS):
                w = wxy[cx][cy] * wz[cz]
                f0 = f0 + rows[p][2 * ci][s] * w
                f1 = f1 + rows[p][2 * ci + 1][s] * w
            peb[pl.ds(2 * l * CHUNK + i * LANES, LANES)] = f0
            peb[pl.ds((2 * l + 1) * CHUNK + i * LANES, LANES)] = f1
            return c

        lax.fori_loop(0, NGROUP, gbody, 0, unroll=False)

    def chunk_body(ch, c):
        off = base + ch * CHUNK
        for d in range(3):
            pltpu.sync_copy(xt_hbm.at[pl.ds(d * N_POINTS + off, CHUNK)],
                            xb.at[d])
        compute_idx(0, 0)
        cur = fire(0)
        nxt = None
        for l in range(N_LEVELS):
            p = l & 1
            if l + 1 < N_LEVELS:
                compute_idx(l + 1, 1 - p)
                nxt = fire(1 - p)
            for cp in cur:
                cp.wait()
            interp(l, p)
            cur = nxt
        pltpu.sync_copy(peb, pe_hbm.at[pl.ds((gcb + ch) * PE_CHUNK, PE_CHUNK)])
        return c

    lax.fori_loop(0, NCHUNK, chunk_body, 0, unroll=False)


def _mlp_body(pe_ref, w0_ref, b0_ref, w1_ref, b1_ref, w2_ref, b2_ref, o_ref):
    nb = MLP_BN // CHUNK
    pe = jnp.concatenate([pe_ref[b] for b in range(nb)], axis=-1)
    dn = (((0,), (0,)), ((), ()))
    h = lax.dot_general(w0_ref[...], pe, dn, preferred_element_type=jnp.float32)
    h = jnp.maximum(h + b0_ref[...], 0.0)
    h = lax.dot_general(w1_ref[...], h, dn, preferred_element_type=jnp.float32)
    h = jnp.maximum(h + b1_ref[...], 0.0)
    z = lax.dot_general(w2_ref[...], h, dn, preferred_element_type=jnp.float32)
    z0 = z[0:1, :] + b2_ref[...]
    o_ref[...] = jnp.maximum(z0, 0.0) + jnp.log1p(jnp.exp(-jnp.abs(z0)))


@jax.jit
def kernel(x, bounding_box, tables, W0, b0, W1, b1, W2, b2):
    xt1 = x.T.reshape(-1)
    inv_ext = 1.0 / (bounding_box[1] - bounding_box[0])
    norm1 = jnp.repeat(
        jnp.concatenate([inv_ext, bounding_box[0]]).astype(jnp.float32), CHUNK)
    tabf = tables.reshape(N_LEVELS, TABLE_SIZE // 128, 128, F_PER_LEVEL)
    tabf = tabf.transpose(0, 1, 3, 2).reshape(-1)

    mesh = plsc.VectorSubcoreMesh(core_axis_name="c", subcore_axis_name="s",
                                  num_cores=NC, num_subcores=NS)
    pe1 = pl.kernel(
        _sc_encode_body,
        out_type=jax.ShapeDtypeStruct((N_POINTS * IN_DIM,), jnp.float32),
        mesh=mesh,
        scratch_types=(
            [
                pltpu.VMEM((3, CHUNK), jnp.float32),
                pltpu.VMEM((6 * CHUNK,), jnp.float32),
                pltpu.VMEM((32, CHUNK), jnp.int32),
                pltpu.VMEM((6, CHUNK), jnp.float32),
                pltpu.VMEM((PE_CHUNK,), jnp.float32),
            ]
            + [pltpu.VMEM((CHUNK,), jnp.float32)
               for _ in range(32)]
            + [pltpu.SemaphoreType.DMA, pltpu.SemaphoreType.DMA]
        ),
        compiler_params=pltpu.CompilerParams(needs_layout_passes=False,
                                             use_tc_tiling_on_sc=False),
    )(xt1, tabf, norm1)

    pe3 = pe1.reshape(N_POINTS // CHUNK, IN_DIM, CHUNK)
    grid = (N_POINTS // MLP_BN,)
    dens = pl.pallas_call(
        _mlp_body,
        grid=grid,
        in_specs=[
            pl.BlockSpec((MLP_BN // CHUNK, IN_DIM, CHUNK), lambda i: (i, 0, 0)),
            pl.BlockSpec((IN_DIM, WIDTH), lambda i: (0, 0)),
            pl.BlockSpec((WIDTH, 1), lambda i: (0, 0)),
            pl.BlockSpec((WIDTH, WIDTH), lambda i: (0, 0)),
            pl.BlockSpec((WIDTH, 1), lambda i: (0, 0)),
            pl.BlockSpec((WIDTH, OUT_DIM), lambda i: (0, 0)),
            pl.BlockSpec((1, 1), lambda i: (0, 0)),
        ],
        out_specs=pl.BlockSpec((1, MLP_BN), lambda i: (0, i)),
        out_shape=jax.ShapeDtypeStruct((1, N_POINTS), jnp.float32),
    )(pe3, W0, b0[:, None], W1, b1[:, None], W2, b2[0:1, None])
    return dens.reshape(N_POINTS)
